# baseline (speedup 1.0000x reference)
import jax
import jax.numpy as jnp
from jax.experimental import pallas as pl

N_NODES = 50000
N_EDGES = 800000
NUM_GRAPHS = 5000
EMB = 32
HID = 64


def _gat_conv(h_in, src, dst, W, a_s, a_d, b, heads, out_ch, concat, N):
    h = (h_in @ W).reshape(N, heads, out_ch)
    alpha_src = (h * a_s).sum(-1)
    alpha_dst = (h * a_d).sum(-1)
    e = alpha_src[src] + alpha_dst[dst]
    e = jax.nn.leaky_relu(e, 0.2)
    e_max = jax.ops.segment_max(e, dst, num_segments=N)
    ex = jnp.exp(e - e_max[dst])
    denom = jax.ops.segment_sum(ex, dst, num_segments=N)
    alpha = ex / (denom[dst] + 1e-16)
    msg = h[src] * alpha[:, :, None]
    out = jax.ops.segment_sum(msg, dst, num_segments=N)
    if concat:
        out = out.reshape(N, heads * out_ch)
    else:
        out = out.mean(axis=1)
    return out + b


def _final_kernel(pooled_ref, fcW_ref, fcb_ref, out_ref):
    out_ref[...] = jax.nn.sigmoid(pooled_ref[...] @ fcW_ref[...] + fcb_ref[...])


def kernel(x, edge_index, batch, emb, W1, att_src1, att_dst1, b1, W2, att_src2, att_dst2, b2, fcW, fcb):
    N = x.shape[0]
    loop = jnp.arange(N, dtype=edge_index.dtype)
    src = jnp.concatenate([edge_index[0], loop])
    dst = jnp.concatenate([edge_index[1], loop])
    h = jnp.take(emb, x, axis=0)
    h = jax.nn.elu(_gat_conv(h, src, dst, W1, att_src1, att_dst1, b1, 4, HID, True, N))
    h = jax.nn.elu(_gat_conv(h, src, dst, W2, att_src2, att_dst2, b2, 1, HID, False, N))
    summed = jax.ops.segment_sum(h, batch, num_segments=NUM_GRAPHS)
    counts = jax.ops.segment_sum(jnp.ones((N,), dtype=h.dtype), batch, num_segments=NUM_GRAPHS)
    pooled = summed / jnp.maximum(counts, 1.0)[:, None]
    return pl.pallas_call(
        _final_kernel,
        out_shape=jax.ShapeDtypeStruct((NUM_GRAPHS, 1), jnp.float32),
    )(pooled, fcW, fcb)


# speedup vs baseline: 15.4333x; 15.4333x over previous
import dataclasses
import functools

import jax
import jax.numpy as jnp
from jax import lax
from jax.experimental import pallas as pl
from jax.experimental.pallas import tpu as pltpu
from jax.experimental.pallas import tpu_sc as plsc

N_NODES = 50000
NUM_GRAPHS = 5000
EMB = 32
HID = 64

_NC = 2
_NS = 16
_L = 16
_NSUB = _NC * _NS

N_TAB = 50176
E_TOT = 850000
SLICE = 26624
E_PAD = _NSUB * SLICE
HALF = SLICE // 2
GCH = 128
WIN = 1024
N_MERGE = 51200


def _sc_compiler_params():
    cp = pltpu.CompilerParams()
    if "needs_layout_passes" in pltpu.CompilerParams.__dataclass_fields__:
        cp = dataclasses.replace(cp, needs_layout_passes=False)
    return cp


_mesh_cache = []


def _MESH():
    if not _mesh_cache:
        _mesh_cache.append(
            plsc.VectorSubcoreMesh(core_axis_name="c", subcore_axis_name="s"))
    return _mesh_cache[0]


def _gather_rows_sc(table, idx, width):
    n = idx.shape[0]
    per = n // _NSUB
    gc2 = 112
    assert per % gc2 == 0

    @functools.partial(
        pl.kernel,
        mesh=_MESH(),
        out_type=jax.ShapeDtypeStruct((n, width), jnp.float32),
        scratch_types=[
            pltpu.VMEM((gc2,), jnp.int32),
            pltpu.VMEM((gc2, width), jnp.float32),
            pltpu.SemaphoreType.DMA,
        ],
        compiler_params=_sc_compiler_params(),
    )
    def k(tab_hbm, idx_hbm, out_hbm, idx_v, rows_v, sem):
        cid = lax.axis_index("c")
        sid = lax.axis_index("s")
        base = (cid * _NS + sid) * per

        @pl.loop(0, per, step=gc2)
        def _(i):
            pltpu.sync_copy(idx_hbm.at[pl.ds(base + i, gc2)], idx_v)
            pltpu.async_copy(tab_hbm.at[idx_v], rows_v, sem).wait()
            pltpu.sync_copy(rows_v, out_hbm.at[pl.ds(base + i, gc2)])

    return k(table, idx)


TROW = 128


def _edge_pass_sc(srcs, dsts, tab_t, tab_d, heads):
    @functools.partial(
        pl.kernel,
        mesh=_MESH(),
        out_type=[
            jax.ShapeDtypeStruct((_NSUB, 2, heads, WIN // 2, TROW), jnp.float32),
            jax.ShapeDtypeStruct((_NSUB, 2, WIN // 8, TROW), jnp.float32),
            jax.ShapeDtypeStruct((_NSUB, 2, _L), jnp.int32),
        ],
        scratch_types=[
            pltpu.VMEM((GCH,), jnp.int32),
            pltpu.VMEM((GCH,), jnp.int32),
            pltpu.VMEM((1, GCH), jnp.int32),
            pltpu.VMEM((WIN // 8, TROW), jnp.float32),
            pltpu.VMEM((GCH, TROW), jnp.float32),
            pltpu.VMEM((WIN // 2, TROW), jnp.float32),
            pltpu.VMEM((WIN // 8, TROW), jnp.float32),
            pltpu.VMEM((_L,), jnp.int32),
            pltpu.SemaphoreType.DMA,
        ],
        compiler_params=_sc_compiler_params(),
    )
    def k(src_hbm, dst_hbm, t_hbm, td_hbm, outw, outd, outf,
          src_c, dst_c, gix, dwin, gslab, acc, dacc, fbuf, sem):
        cid = lax.axis_index("c")
        sid = lax.axis_index("s")
        sub = cid * _NS + sid

        iota16 = lax.iota(jnp.int32, _L)
        zero16 = jnp.zeros((_L,), jnp.float32)
        hmask = iota16 < heads

        for half in range(2):
            hbase = sub * SLICE + half * HALF
            pltpu.sync_copy(dst_hbm.at[pl.ds(hbase, GCH)], dst_c)
            first = jnp.bitwise_and(dst_c[pl.ds(0, _L)][0], jnp.int32(-64))
            fbuf[...] = jnp.full((_L,), first, jnp.int32)
            pltpu.sync_copy(fbuf, outf.at[sub].at[half])
            first8 = pl.multiple_of(first // 8, 8)
            pltpu.sync_copy(td_hbm.at[pl.ds(first8, WIN // 8)], dwin)

            @pl.loop(0, WIN // 8)
            def _(r):
                for kk in range(TROW // _L):
                    dacc[r, pl.ds(kk * _L, _L)] = zero16

            for hd in range(heads):
                @pl.loop(0, WIN // 2)
                def _(r):
                    for kk in range(TROW // _L):
                        acc[r, pl.ds(kk * _L, _L)] = zero16

                @pl.loop(0, HALF, step=GCH)
                def _(i):
                    off = hbase + i
                    pltpu.sync_copy(src_hbm.at[pl.ds(off, GCH)], src_c)
                    pltpu.sync_copy(dst_hbm.at[pl.ds(off, GCH)], dst_c)

                    @pl.loop(0, GCH, step=_L)
                    def _(g):
                        sv = src_c[pl.ds(g, _L)]
                        plsc.store_scatter(
                            gix, [jnp.zeros((_L,), jnp.int32), g + iota16],
                            sv + hd * N_TAB)

                    pltpu.async_copy(t_hbm.at[gix.at[0]], gslab, sem).wait()

                    @pl.loop(0, GCH, step=_L)
                    def _(g):
                        relv = jnp.clip(dst_c[pl.ds(g, _L)] - first, 0, WIN - 1)
                        for j in range(_L):
                            rel_j = relv[j]
                            dr8 = jnp.full((_L,), rel_j // 8, jnp.int32)
                            dc = (rel_j % 8) * _L + iota16
                            drow = plsc.load_gather(dwin, [dr8, dc])
                            zr = gslab[g + j, pl.ds(0, _L)] + drow
                            exr = jnp.exp(jnp.where(zr < 0, zr * 0.2, zr))
                            if hd == 0:
                                plsc.addupdate_scatter(dacc, [dr8, dc], exr,
                                                       mask=hmask)
                            exb = jnp.full((_L,), exr[hd], jnp.float32)
                            ar2 = jnp.full((_L,), rel_j // 2, jnp.int32)
                            cbase = (rel_j % 2) * HID
                            for kk in range(HID // _L):
                                val = gslab[g + j, pl.ds(_L + kk * _L, _L)] * exb
                                plsc.addupdate_scatter(
                                    acc, [ar2, cbase + kk * _L + iota16], val)

                pltpu.sync_copy(acc, outw.at[sub].at[half].at[hd])

            pltpu.sync_copy(dacc, outd.at[sub].at[half])

    return k(srcs, dsts, tab_t, tab_d)


def _merge_tc(wins, dens, firsts, heads):
    nwin = _NSUB * 2

    def body(firsts_ref, win_ref, den_ref, num_ref, den_out_ref):
        h = pl.program_id(0)
        w = pl.program_id(1)
        first = firsts_ref[w // 2, w % 2, 0]
        sw = pl.multiple_of(first // 2, 8)
        sd = pl.multiple_of(first // 8, 8)

        @pl.when(w == 0)
        def _():
            num_ref[...] = jnp.zeros_like(num_ref)

        num_ref[0, pl.ds(sw, WIN // 2), :] = (
            num_ref[0, pl.ds(sw, WIN // 2), :] + win_ref[0, 0, 0])

        @pl.when(jnp.logical_and(h == 0, w == 0))
        def _():
            den_out_ref[...] = jnp.zeros_like(den_out_ref)

        @pl.when(h == 0)
        def _():
            den_out_ref[pl.ds(sd, WIN // 8), :] = (
                den_out_ref[pl.ds(sd, WIN // 8), :] + den_ref[0, 0])

    return pl.pallas_call(
        body,
        grid=(heads, nwin),
        in_specs=[
            pl.BlockSpec(memory_space=pltpu.SMEM),
            pl.BlockSpec((1, 1, 1, WIN // 2, TROW),
                         lambda h, w: (w // 2, w % 2, h, 0, 0)),
            pl.BlockSpec((1, 1, WIN // 8, TROW),
                         lambda h, w: (w // 2, w % 2, 0, 0)),
        ],
        out_specs=[
            pl.BlockSpec((1, N_MERGE // 2, TROW), lambda h, w: (h, 0, 0)),
            pl.BlockSpec((N_MERGE // 8, TROW), lambda h, w: (0, 0)),
        ],
        out_shape=[
            jax.ShapeDtypeStruct((heads, N_MERGE // 2, TROW), jnp.float32),
            jax.ShapeDtypeStruct((N_MERGE // 8, TROW), jnp.float32),
        ],
    )(firsts, wins, dens)


_BLK = 512


def _prep1_tc(hx, W1, att_src1, att_dst1):
    def body(hx_ref, w_ref, as_ref, ad_ref, t_ref, td_ref):
        h1 = hx_ref[...][:, :EMB] @ w_ref[...]
        i = pl.program_id(0)
        absrow = i * _BLK + jax.lax.broadcasted_iota(jnp.int32, (_BLK, 1), 0)
        a_ss, a_ds = [], []
        for hh in range(4):
            h1h = h1[:, hh * HID:(hh + 1) * HID]
            a_ss.append((h1h * as_ref[0, hh][None, :]).sum(-1, keepdims=True))
            a_ds.append((h1h * ad_ref[0, hh][None, :]).sum(-1, keepdims=True))
        a16 = jnp.concatenate(
            a_ss + [jnp.zeros((_BLK, 12), jnp.float32)], axis=1)
        for hh in range(4):
            h1h = h1[:, hh * HID:(hh + 1) * HID]
            t_ref[hh] = jnp.concatenate(
                [a16, h1h, jnp.zeros((_BLK, TROW - _L - HID), jnp.float32)],
                axis=1)
        a_d = jnp.concatenate(a_ds, axis=1)
        a_d = jnp.where(absrow >= N_NODES, -1e30, a_d)
        td_ref[...] = jnp.concatenate(
            [a_d, jnp.zeros((_BLK, 12), jnp.float32)], axis=1)

    return pl.pallas_call(
        body,
        grid=(N_TAB // _BLK,),
        in_specs=[
            pl.BlockSpec((_BLK, TROW), lambda i: (i, 0)),
            pl.BlockSpec((EMB, 4 * HID), lambda i: (0, 0)),
            pl.BlockSpec((1, 4, HID), lambda i: (0, 0, 0)),
            pl.BlockSpec((1, 4, HID), lambda i: (0, 0, 0)),
        ],
        out_specs=[
            pl.BlockSpec((4, _BLK, TROW), lambda i: (0, i, 0)),
            pl.BlockSpec((_BLK, _L), lambda i: (i, 0)),
        ],
        out_shape=[
            jax.ShapeDtypeStruct((4, N_TAB, TROW), jnp.float32),
            jax.ShapeDtypeStruct((N_TAB, _L), jnp.float32),
        ],
    )(hx, W1, att_src1, att_dst1)


def _combine1_prep2_tc(num1, den1, b1, W2, att_src2, att_dst2):
    def body(num_ref, den_ref, b1_ref, w2_ref, as_ref, ad_ref,
             t2_ref, td2_ref):
        den = den_ref[...]
        parts = []
        for hh in range(4):
            parts.append(num_ref[hh] / (den[:, hh:hh + 1] + 1e-16))
        h1 = jnp.concatenate(parts, axis=1) + b1_ref[...]
        h1 = jnp.where(h1 > 0, h1, (jnp.exp(h1) - 1.0))
        h2 = h1 @ w2_ref[...]
        a_s = (h2 * as_ref[0, 0][None, :]).sum(-1, keepdims=True)
        a_d = (h2 * ad_ref[0, 0][None, :]).sum(-1, keepdims=True)
        i = pl.program_id(0)
        absrow = i * _BLK + jax.lax.broadcasted_iota(jnp.int32, (_BLK, 1), 0)
        a_d = jnp.where(absrow >= N_NODES, -1e30, a_d)
        t2_ref[...] = jnp.concatenate(
            [a_s, jnp.zeros((_BLK, 15), jnp.float32), h2,
             jnp.zeros((_BLK, TROW - _L - HID), jnp.float32)], axis=1)
        td2_ref[...] = jnp.concatenate(
            [a_d, jnp.zeros((_BLK, 15), jnp.float32)], axis=1)

    return pl.pallas_call(
        body,
        grid=(N_TAB // _BLK,),
        in_specs=[
            pl.BlockSpec((4, _BLK, HID), lambda i: (0, i, 0)),
            pl.BlockSpec((_BLK, _L), lambda i: (i, 0)),
            pl.BlockSpec((1, 4 * HID), lambda i: (0, 0)),
            pl.BlockSpec((4 * HID, HID), lambda i: (0, 0)),
            pl.BlockSpec((1, 1, HID), lambda i: (0, 0, 0)),
            pl.BlockSpec((1, 1, HID), lambda i: (0, 0, 0)),
        ],
        out_specs=[
            pl.BlockSpec((_BLK, TROW), lambda i: (i, 0)),
            pl.BlockSpec((_BLK, _L), lambda i: (i, 0)),
        ],
        out_shape=[
            jax.ShapeDtypeStruct((N_TAB, TROW), jnp.float32),
            jax.ShapeDtypeStruct((N_TAB, _L), jnp.float32),
        ],
    )(num1, den1, b1, W2, att_src2, att_dst2)


AUG = 80


def _combine2_tc(num2, den2, b2):
    def body(num_ref, den_ref, b2_ref, aug_ref):
        num = num_ref[0]
        den = den_ref[...][:, :1]
        h = num / (den + 1e-16) + b2_ref[...]
        h = jnp.where(h > 0, h, (jnp.exp(h) - 1.0))
        ones = jnp.ones((_BLK, 1), jnp.float32)
        pad = jnp.zeros((_BLK, AUG - HID - 1), jnp.float32)
        aug_ref[...] = jnp.concatenate([h, ones, pad], axis=1)

    return pl.pallas_call(
        body,
        grid=(N_TAB // _BLK,),
        in_specs=[
            pl.BlockSpec((1, _BLK, HID), lambda i: (0, i, 0)),
            pl.BlockSpec((_BLK, _L), lambda i: (i, 0)),
            pl.BlockSpec((1, HID), lambda i: (0, 0)),
        ],
        out_specs=pl.BlockSpec((_BLK, AUG), lambda i: (i, 0)),
        out_shape=jax.ShapeDtypeStruct((N_TAB, AUG), jnp.float32),
    )(num2, den2, b2)


POOL_CHUNK = 128
POOL_CHUNKS_PER_SUB = 13
POOL_ROWS_PER_SUB = POOL_CHUNK * POOL_CHUNKS_PER_SUB
N_POOL = _NSUB * POOL_ROWS_PER_SUB
LW = 512
G_PAD2 = NUM_GRAPHS + 632


def _pool_sc(h_aug, batch_pad):
    @functools.partial(
        pl.kernel,
        mesh=_MESH(),
        out_type=[
            jax.ShapeDtypeStruct((_NSUB, LW, AUG), jnp.float32),
            jax.ShapeDtypeStruct((_NSUB, 16), jnp.int32),
        ],
        scratch_types=[
            pltpu.VMEM((POOL_CHUNK, AUG), jnp.float32),
            pltpu.VMEM((POOL_CHUNK,), jnp.int32),
            pltpu.VMEM((LW, AUG), jnp.float32),
            pltpu.VMEM((16,), jnp.int32),
            pltpu.SemaphoreType.DMA,
        ],
        compiler_params=_sc_compiler_params(),
    )
    def k(h_hbm, b_hbm, outw_hbm, outf_hbm, rows_v, idx_v, local, fidx, sem):
        cid = lax.axis_index("c")
        sid = lax.axis_index("s")
        wid = cid * _NS + sid
        base = wid * POOL_ROWS_PER_SUB

        zero16 = jnp.zeros((_L,), jnp.float32)

        @pl.loop(0, LW)
        def _(r):
            for kk in range(AUG // _L):
                local[r, pl.ds(kk * _L, _L)] = zero16

        pltpu.sync_copy(b_hbm.at[pl.ds(base, POOL_CHUNK)], idx_v)
        first_al = jnp.bitwise_and(idx_v[pl.ds(0, _L)][0], jnp.int32(-8))
        fidx[...] = jnp.full((_L,), first_al, jnp.int32)

        iota16 = lax.iota(jnp.int32, _L)

        @pl.loop(0, POOL_CHUNKS_PER_SUB)
        def _(i):
            off = base + i * POOL_CHUNK
            pltpu.sync_copy(b_hbm.at[pl.ds(off, POOL_CHUNK)], idx_v)
            pltpu.sync_copy(h_hbm.at[pl.ds(off, POOL_CHUNK)], rows_v)

            @pl.loop(0, POOL_CHUNK, step=_L)
            def _(j):
                grelv = jnp.clip(idx_v[pl.ds(j, _L)] - first_al, 0, LW - 1)
                for t in range(_L):
                    rowv = jnp.full((_L,), grelv[t], jnp.int32)
                    for kk in range(AUG // _L):
                        vals = rows_v[j + t, pl.ds(kk * _L, _L)]
                        plsc.addupdate_scatter(local, [rowv, iota16 + kk * _L], vals)

        pltpu.sync_copy(local, outw_hbm.at[wid])
        pltpu.sync_copy(fidx, outf_hbm.at[wid])

    return k(h_aug, batch_pad)


def _merge_final_tc(firsts_ref, win_ref, fcW_ref, fcb_ref, out_ref, acc_ref):
    i = pl.program_id(0)

    @pl.when(i == 0)
    def _():
        acc_ref[...] = jnp.zeros_like(acc_ref)

    start = firsts_ref[i, 0]
    acc_ref[pl.ds(start, LW), :] = acc_ref[pl.ds(start, LW), :] + win_ref[0]

    @pl.when(i == pl.num_programs(0) - 1)
    def _():
        p = acc_ref[:NUM_GRAPHS, :]
        cnt = jnp.maximum(p[:, HID], 1.0)
        pooled = p[:, :HID] / cnt[:, None]
        out_ref[...] = jax.nn.sigmoid(pooled @ fcW_ref[...] + fcb_ref[0])


def _pool_and_head(h_aug_core, batch, fcW, fcb):
    h_aug = jnp.concatenate(
        [h_aug_core[:N_NODES],
         jnp.zeros((N_POOL - N_NODES, AUG), jnp.float32)], axis=0)
    b32 = batch.astype(jnp.int32)
    batch_pad = jnp.concatenate(
        [b32, jnp.full((N_POOL - N_NODES,), NUM_GRAPHS, jnp.int32)])
    wins, firsts = _pool_sc(h_aug, batch_pad)
    return pl.pallas_call(
        _merge_final_tc,
        grid=(_NSUB,),
        in_specs=[
            pl.BlockSpec(memory_space=pltpu.SMEM),
            pl.BlockSpec((1, LW, AUG), lambda i: (i, 0, 0)),
            pl.BlockSpec((HID, 1), lambda i: (0, 0)),
            pl.BlockSpec(memory_space=pltpu.SMEM),
        ],
        out_specs=pl.BlockSpec((NUM_GRAPHS, 1), lambda i: (0, 0)),
        out_shape=jax.ShapeDtypeStruct((NUM_GRAPHS, 1), jnp.float32),
        scratch_shapes=[pltpu.VMEM((G_PAD2, AUG), jnp.float32)],
    )(firsts, wins, fcW, fcb)


def kernel(x, edge_index, batch, emb, W1, att_src1, att_dst1, b1,
           W2, att_src2, att_dst2, b2, fcW, fcb):
    N = x.shape[0]
    loop = jnp.arange(N, dtype=jnp.int32)
    src = jnp.concatenate([edge_index[0].astype(jnp.int32), loop])
    dst = jnp.concatenate([edge_index[1].astype(jnp.int32), loop])
    dsts, srcs = lax.sort_key_val(dst, src)
    dsts = jnp.concatenate(
        [dsts, jnp.full((E_PAD - E_TOT,), N_TAB - 1, jnp.int32)])
    srcs = jnp.concatenate([srcs, jnp.zeros((E_PAD - E_TOT,), jnp.int32)])

    x_pad = jnp.concatenate(
        [x.astype(jnp.int32), jnp.zeros((N_TAB - N,), jnp.int32)])
    emb128 = jnp.pad(emb, ((0, 0), (0, TROW - EMB)))
    hx = _gather_rows_sc(emb128, x_pad, TROW)

    t1, td1 = _prep1_tc(hx, W1, att_src1, att_dst1)
    t1 = t1.reshape(4 * N_TAB, TROW)
    td1p = jnp.pad(td1, ((0, WIN), (0, 0))).reshape((N_TAB + WIN) // 8, TROW)
    wins1, dens1, firsts1 = _edge_pass_sc(srcs, dsts, t1, td1p, 4)
    num1p, den1p = _merge_tc(wins1, dens1, firsts1, 4)
    num1 = num1p.reshape(4, N_MERGE, HID)
    den1 = den1p.reshape(N_MERGE, _L)

    t2, td2 = _combine1_prep2_tc(
        num1[:, :N_TAB, :], den1[:N_TAB], b1.reshape(1, -1), W2,
        att_src2, att_dst2)
    td2p = jnp.pad(td2, ((0, WIN), (0, 0))).reshape((N_TAB + WIN) // 8, TROW)
    wins2, dens2, firsts2 = _edge_pass_sc(srcs, dsts, t2, td2p, 1)
    num2p, den2p = _merge_tc(wins2, dens2, firsts2, 1)
    num2 = num2p.reshape(1, N_MERGE, HID)
    den2 = den2p.reshape(N_MERGE, _L)

    h_aug_core = _combine2_tc(num2[:, :N_TAB, :], den2[:N_TAB], b2.reshape(1, -1))
    return _pool_and_head(h_aug_core, batch, fcW, fcb)


# speedup vs baseline: 16.6716x; 1.0802x over previous
import dataclasses
import functools

import jax
import jax.numpy as jnp
from jax import lax
from jax.experimental import pallas as pl
from jax.experimental.pallas import tpu as pltpu
from jax.experimental.pallas import tpu_sc as plsc

N_NODES = 50000
NUM_GRAPHS = 5000
EMB = 32
HID = 64

_NC = 2
_NS = 16
_L = 16
_NSUB = _NC * _NS

N_TAB = 50176
E_TOT = 850000
SLICE = 26624
E_PAD = _NSUB * SLICE
HALF = SLICE // 2
GCH = 128
CH2 = 64
NCH = HALF // CH2
WIN = 1024
N_MERGE = 51200


def _sc_compiler_params():
    cp = pltpu.CompilerParams()
    if "needs_layout_passes" in pltpu.CompilerParams.__dataclass_fields__:
        cp = dataclasses.replace(cp, needs_layout_passes=False)
    return cp


_mesh_cache = []


def _MESH():
    if not _mesh_cache:
        _mesh_cache.append(
            plsc.VectorSubcoreMesh(core_axis_name="c", subcore_axis_name="s"))
    return _mesh_cache[0]


def _gather_rows_sc(table, idx, width):
    n = idx.shape[0]
    per = n // _NSUB
    gc2 = 112
    assert per % gc2 == 0

    @functools.partial(
        pl.kernel,
        mesh=_MESH(),
        out_type=jax.ShapeDtypeStruct((n, width), jnp.float32),
        scratch_types=[
            pltpu.VMEM((gc2,), jnp.int32),
            pltpu.VMEM((gc2, width), jnp.float32),
            pltpu.SemaphoreType.DMA,
        ],
        compiler_params=_sc_compiler_params(),
    )
    def k(tab_hbm, idx_hbm, out_hbm, idx_v, rows_v, sem):
        cid = lax.axis_index("c")
        sid = lax.axis_index("s")
        base = (cid * _NS + sid) * per

        @pl.loop(0, per, step=gc2)
        def _(i):
            pltpu.sync_copy(idx_hbm.at[pl.ds(base + i, gc2)], idx_v)
            pltpu.async_copy(tab_hbm.at[idx_v], rows_v, sem).wait()
            pltpu.sync_copy(rows_v, out_hbm.at[pl.ds(base + i, gc2)])

    return k(table, idx)


TROW = 128


def _edge_pass_sc(srcs, dsts, tab_t, tab_d, heads):
    @functools.partial(
        pl.kernel,
        mesh=_MESH(),
        out_type=[
            jax.ShapeDtypeStruct((_NSUB, 2, heads, WIN // 2, TROW), jnp.float32),
            jax.ShapeDtypeStruct((_NSUB, 2, WIN // 8, TROW), jnp.float32),
            jax.ShapeDtypeStruct((_NSUB, 2, _L), jnp.int32),
        ],
        scratch_types=[
            pltpu.VMEM((CH2,), jnp.int32),
            pltpu.VMEM((CH2,), jnp.int32),
            pltpu.VMEM((CH2,), jnp.int32),
            pltpu.VMEM((1, CH2), jnp.int32),
            pltpu.VMEM((1, CH2), jnp.int32),
            pltpu.VMEM((WIN // 8, TROW), jnp.float32),
            pltpu.VMEM((CH2, TROW), jnp.float32),
            pltpu.VMEM((CH2, TROW), jnp.float32),
            pltpu.VMEM((WIN // 2, TROW), jnp.float32),
            pltpu.VMEM((WIN // 8, TROW), jnp.float32),
            pltpu.VMEM((_L,), jnp.int32),
            pltpu.SemaphoreType.DMA,
        ],
        compiler_params=_sc_compiler_params(),
    )
    def k(src_hbm, dst_hbm, t_hbm, td_hbm, outw, outd, outf,
          src_c, dst_c0, dst_c1, gix0, gix1, dwin, gslab0, gslab1,
          acc, dacc, fbuf, sem):
        cid = lax.axis_index("c")
        sid = lax.axis_index("s")
        sub = cid * _NS + sid

        iota16 = lax.iota(jnp.int32, _L)
        zero16 = jnp.zeros((_L,), jnp.float32)
        hmask = iota16 < heads

        def prefetch(off, dst_c, gix, gslab, hd):
            pltpu.sync_copy(src_hbm.at[pl.ds(off, CH2)], src_c)
            pltpu.sync_copy(dst_hbm.at[pl.ds(off, CH2)], dst_c)

            @pl.loop(0, CH2, step=_L)
            def _(g):
                sv = src_c[pl.ds(g, _L)]
                plsc.store_scatter(
                    gix, [jnp.zeros((_L,), jnp.int32), g + iota16],
                    sv + hd * N_TAB)

            pltpu.async_copy(t_hbm.at[gix.at[0]], gslab, sem)

        def wait_gather(gix, gslab):
            pltpu.make_async_copy(t_hbm.at[gix.at[0]], gslab, sem).wait()

        def compute(dst_c, gslab, first, hd, heads):
            @pl.loop(0, CH2, step=_L)
            def _(g):
                relv = jnp.clip(dst_c[pl.ds(g, _L)] - first, 0, WIN - 1)
                for j in range(_L):
                    rel_j = relv[j]
                    dr8 = jnp.full((_L,), rel_j // 8, jnp.int32)
                    dc = (rel_j % 8) * _L + iota16
                    drow = plsc.load_gather(dwin, [dr8, dc])
                    zr = gslab[g + j, pl.ds(0, _L)] + drow
                    exr = jnp.exp(jnp.where(zr < 0, zr * 0.2, zr))
                    if hd == 0:
                        plsc.addupdate_scatter(dacc, [dr8, dc], exr,
                                               mask=iota16 < heads)
                    exb = jnp.full((_L,), exr[hd], jnp.float32)
                    ar2 = jnp.full((_L,), rel_j // 2, jnp.int32)
                    cbase = (rel_j % 2) * HID
                    for kk in range(HID // _L):
                        val = gslab[g + j, pl.ds(_L + kk * _L, _L)] * exb
                        plsc.addupdate_scatter(
                            acc, [ar2, cbase + kk * _L + iota16], val)

        @pl.loop(0, 2)
        def _(half):
            hbase = sub * SLICE + half * HALF
            pltpu.sync_copy(dst_hbm.at[pl.ds(hbase, CH2)], dst_c0)
            first = jnp.bitwise_and(dst_c0[pl.ds(0, _L)][0], jnp.int32(-64))
            fbuf[...] = jnp.full((_L,), first, jnp.int32)
            pltpu.sync_copy(fbuf, outf.at[sub].at[half])
            first8 = pl.multiple_of(first // 8, 8)
            pltpu.sync_copy(td_hbm.at[pl.ds(first8, WIN // 8)], dwin)

            @pl.loop(0, WIN // 8)
            def _(r):
                for kk in range(TROW // _L):
                    dacc[r, pl.ds(kk * _L, _L)] = zero16

            for hd in range(heads):
                @pl.loop(0, WIN // 2)
                def _(r):
                    for kk in range(TROW // _L):
                        acc[r, pl.ds(kk * _L, _L)] = zero16

                prefetch(hbase, dst_c0, gix0, gslab0, hd)

                @pl.loop(0, NCH, step=2)
                def _(i):
                    off = hbase + i * CH2
                    wait_gather(gix0, gslab0)
                    prefetch(off + CH2, dst_c1, gix1, gslab1, hd)
                    compute(dst_c0, gslab0, first, hd, heads)
                    wait_gather(gix1, gslab1)

                    @pl.when(i + 2 < NCH)
                    def _():
                        prefetch(off + 2 * CH2, dst_c0, gix0, gslab0, hd)

                    compute(dst_c1, gslab1, first, hd, heads)

                pltpu.sync_copy(acc, outw.at[sub].at[half].at[hd])

            pltpu.sync_copy(dacc, outd.at[sub].at[half])

    return k(srcs, dsts, tab_t, tab_d)


def _merge_tc(wins, dens, firsts, heads):
    nwin = _NSUB * 2

    def body(firsts_ref, win_ref, den_ref, num_ref, den_out_ref):
        h = pl.program_id(0)
        w = pl.program_id(1)
        first = firsts_ref[w // 2, w % 2, 0]
        sw = pl.multiple_of(first // 2, 8)
        sd = pl.multiple_of(first // 8, 8)

        @pl.when(w == 0)
        def _():
            num_ref[...] = jnp.zeros_like(num_ref)

        num_ref[0, pl.ds(sw, WIN // 2), :] = (
            num_ref[0, pl.ds(sw, WIN // 2), :] + win_ref[0, 0, 0])

        @pl.when(jnp.logical_and(h == 0, w == 0))
        def _():
            den_out_ref[...] = jnp.zeros_like(den_out_ref)

        @pl.when(h == 0)
        def _():
            den_out_ref[pl.ds(sd, WIN // 8), :] = (
                den_out_ref[pl.ds(sd, WIN // 8), :] + den_ref[0, 0])

    return pl.pallas_call(
        body,
        grid=(heads, nwin),
        in_specs=[
            pl.BlockSpec(memory_space=pltpu.SMEM),
            pl.BlockSpec((1, 1, 1, WIN // 2, TROW),
                         lambda h, w: (w // 2, w % 2, h, 0, 0)),
            pl.BlockSpec((1, 1, WIN // 8, TROW),
                         lambda h, w: (w // 2, w % 2, 0, 0)),
        ],
        out_specs=[
            pl.BlockSpec((1, N_MERGE // 2, TROW), lambda h, w: (h, 0, 0)),
            pl.BlockSpec((N_MERGE // 8, TROW), lambda h, w: (0, 0)),
        ],
        out_shape=[
            jax.ShapeDtypeStruct((heads, N_MERGE // 2, TROW), jnp.float32),
            jax.ShapeDtypeStruct((N_MERGE // 8, TROW), jnp.float32),
        ],
    )(firsts, wins, dens)


_BLK = 512


def _prep1_tc(hx, W1, att_src1, att_dst1):
    def body(hx_ref, w_ref, as_ref, ad_ref, t_ref, td_ref):
        h1 = hx_ref[...][:, :EMB] @ w_ref[...]
        i = pl.program_id(0)
        absrow = i * _BLK + jax.lax.broadcasted_iota(jnp.int32, (_BLK, 1), 0)
        a_ss, a_ds = [], []
        for hh in range(4):
            h1h = h1[:, hh * HID:(hh + 1) * HID]
            a_ss.append((h1h * as_ref[0, hh][None, :]).sum(-1, keepdims=True))
            a_ds.append((h1h * ad_ref[0, hh][None, :]).sum(-1, keepdims=True))
        a16 = jnp.concatenate(
            a_ss + [jnp.zeros((_BLK, 12), jnp.float32)], axis=1)
        for hh in range(4):
            h1h = h1[:, hh * HID:(hh + 1) * HID]
            t_ref[hh] = jnp.concatenate(
                [a16, h1h, jnp.zeros((_BLK, TROW - _L - HID), jnp.float32)],
                axis=1)
        a_d = jnp.concatenate(a_ds, axis=1)
        a_d = jnp.where(absrow >= N_NODES, -1e30, a_d)
        td_ref[...] = jnp.concatenate(
            [a_d, jnp.zeros((_BLK, 12), jnp.float32)], axis=1)

    return pl.pallas_call(
        body,
        grid=(N_TAB // _BLK,),
        in_specs=[
            pl.BlockSpec((_BLK, TROW), lambda i: (i, 0)),
            pl.BlockSpec((EMB, 4 * HID), lambda i: (0, 0)),
            pl.BlockSpec((1, 4, HID), lambda i: (0, 0, 0)),
            pl.BlockSpec((1, 4, HID), lambda i: (0, 0, 0)),
        ],
        out_specs=[
            pl.BlockSpec((4, _BLK, TROW), lambda i: (0, i, 0)),
            pl.BlockSpec((_BLK, _L), lambda i: (i, 0)),
        ],
        out_shape=[
            jax.ShapeDtypeStruct((4, N_TAB, TROW), jnp.float32),
            jax.ShapeDtypeStruct((N_TAB, _L), jnp.float32),
        ],
    )(hx, W1, att_src1, att_dst1)


def _combine1_prep2_tc(num1, den1, b1, W2, att_src2, att_dst2):
    def body(num_ref, den_ref, b1_ref, w2_ref, as_ref, ad_ref,
             t2_ref, td2_ref):
        den = den_ref[...]
        parts = []
        for hh in range(4):
            parts.append(num_ref[hh] / (den[:, hh:hh + 1] + 1e-16))
        h1 = jnp.concatenate(parts, axis=1) + b1_ref[...]
        h1 = jnp.where(h1 > 0, h1, (jnp.exp(h1) - 1.0))
        h2 = h1 @ w2_ref[...]
        a_s = (h2 * as_ref[0, 0][None, :]).sum(-1, keepdims=True)
        a_d = (h2 * ad_ref[0, 0][None, :]).sum(-1, keepdims=True)
        i = pl.program_id(0)
        absrow = i * _BLK + jax.lax.broadcasted_iota(jnp.int32, (_BLK, 1), 0)
        a_d = jnp.where(absrow >= N_NODES, -1e30, a_d)
        t2_ref[...] = jnp.concatenate(
            [a_s, jnp.zeros((_BLK, 15), jnp.float32), h2,
             jnp.zeros((_BLK, TROW - _L - HID), jnp.float32)], axis=1)
        td2_ref[...] = jnp.concatenate(
            [a_d, jnp.zeros((_BLK, 15), jnp.float32)], axis=1)

    return pl.pallas_call(
        body,
        grid=(N_TAB // _BLK,),
        in_specs=[
            pl.BlockSpec((4, _BLK, HID), lambda i: (0, i, 0)),
            pl.BlockSpec((_BLK, _L), lambda i: (i, 0)),
            pl.BlockSpec((1, 4 * HID), lambda i: (0, 0)),
            pl.BlockSpec((4 * HID, HID), lambda i: (0, 0)),
            pl.BlockSpec((1, 1, HID), lambda i: (0, 0, 0)),
            pl.BlockSpec((1, 1, HID), lambda i: (0, 0, 0)),
        ],
        out_specs=[
            pl.BlockSpec((_BLK, TROW), lambda i: (i, 0)),
            pl.BlockSpec((_BLK, _L), lambda i: (i, 0)),
        ],
        out_shape=[
            jax.ShapeDtypeStruct((N_TAB, TROW), jnp.float32),
            jax.ShapeDtypeStruct((N_TAB, _L), jnp.float32),
        ],
    )(num1, den1, b1, W2, att_src2, att_dst2)


AUG = 80


def _combine2_tc(num2, den2, b2):
    def body(num_ref, den_ref, b2_ref, aug_ref):
        num = num_ref[0]
        den = den_ref[...][:, :1]
        h = num / (den + 1e-16) + b2_ref[...]
        h = jnp.where(h > 0, h, (jnp.exp(h) - 1.0))
        ones = jnp.ones((_BLK, 1), jnp.float32)
        pad = jnp.zeros((_BLK, AUG - HID - 1), jnp.float32)
        aug_ref[...] = jnp.concatenate([h, ones, pad], axis=1)

    return pl.pallas_call(
        body,
        grid=(N_TAB // _BLK,),
        in_specs=[
            pl.BlockSpec((1, _BLK, HID), lambda i: (0, i, 0)),
            pl.BlockSpec((_BLK, _L), lambda i: (i, 0)),
            pl.BlockSpec((1, HID), lambda i: (0, 0)),
        ],
        out_specs=pl.BlockSpec((_BLK, AUG), lambda i: (i, 0)),
        out_shape=jax.ShapeDtypeStruct((N_TAB, AUG), jnp.float32),
    )(num2, den2, b2)


POOL_CHUNK = 128
POOL_CHUNKS_PER_SUB = 13
POOL_ROWS_PER_SUB = POOL_CHUNK * POOL_CHUNKS_PER_SUB
N_POOL = _NSUB * POOL_ROWS_PER_SUB
LW = 512
G_PAD2 = NUM_GRAPHS + 632


def _pool_sc(h_aug, batch_pad):
    @functools.partial(
        pl.kernel,
        mesh=_MESH(),
        out_type=[
            jax.ShapeDtypeStruct((_NSUB, LW, AUG), jnp.float32),
            jax.ShapeDtypeStruct((_NSUB, 16), jnp.int32),
        ],
        scratch_types=[
            pltpu.VMEM((POOL_CHUNK, AUG), jnp.float32),
            pltpu.VMEM((POOL_CHUNK,), jnp.int32),
            pltpu.VMEM((LW, AUG), jnp.float32),
            pltpu.VMEM((16,), jnp.int32),
            pltpu.SemaphoreType.DMA,
        ],
        compiler_params=_sc_compiler_params(),
    )
    def k(h_hbm, b_hbm, outw_hbm, outf_hbm, rows_v, idx_v, local, fidx, sem):
        cid = lax.axis_index("c")
        sid = lax.axis_index("s")
        wid = cid * _NS + sid
        base = wid * POOL_ROWS_PER_SUB

        zero16 = jnp.zeros((_L,), jnp.float32)

        @pl.loop(0, LW)
        def _(r):
            for kk in range(AUG // _L):
                local[r, pl.ds(kk * _L, _L)] = zero16

        pltpu.sync_copy(b_hbm.at[pl.ds(base, POOL_CHUNK)], idx_v)
        first_al = jnp.bitwise_and(idx_v[pl.ds(0, _L)][0], jnp.int32(-8))
        fidx[...] = jnp.full((_L,), first_al, jnp.int32)

        iota16 = lax.iota(jnp.int32, _L)

        @pl.loop(0, POOL_CHUNKS_PER_SUB)
        def _(i):
            off = base + i * POOL_CHUNK
            pltpu.sync_copy(b_hbm.at[pl.ds(off, POOL_CHUNK)], idx_v)
            pltpu.sync_copy(h_hbm.at[pl.ds(off, POOL_CHUNK)], rows_v)

            @pl.loop(0, POOL_CHUNK, step=_L)
            def _(j):
                grelv = jnp.clip(idx_v[pl.ds(j, _L)] - first_al, 0, LW - 1)
                for t in range(_L):
                    rowv = jnp.full((_L,), grelv[t], jnp.int32)
                    for kk in range(AUG // _L):
                        vals = rows_v[j + t, pl.ds(kk * _L, _L)]
                        plsc.addupdate_scatter(local, [rowv, iota16 + kk * _L], vals)

        pltpu.sync_copy(local, outw_hbm.at[wid])
        pltpu.sync_copy(fidx, outf_hbm.at[wid])

    return k(h_aug, batch_pad)


def _merge_final_tc(firsts_ref, win_ref, fcW_ref, fcb_ref, out_ref, acc_ref):
    i = pl.program_id(0)

    @pl.when(i == 0)
    def _():
        acc_ref[...] = jnp.zeros_like(acc_ref)

    start = firsts_ref[i, 0]
    acc_ref[pl.ds(start, LW), :] = acc_ref[pl.ds(start, LW), :] + win_ref[0]

    @pl.when(i == pl.num_programs(0) - 1)
    def _():
        p = acc_ref[:NUM_GRAPHS, :]
        cnt = jnp.maximum(p[:, HID], 1.0)
        pooled = p[:, :HID] / cnt[:, None]
        out_ref[...] = jax.nn.sigmoid(pooled @ fcW_ref[...] + fcb_ref[0])


def _pool_and_head(h_aug_core, batch, fcW, fcb):
    h_aug = jnp.concatenate(
        [h_aug_core[:N_NODES],
         jnp.zeros((N_POOL - N_NODES, AUG), jnp.float32)], axis=0)
    b32 = batch.astype(jnp.int32)
    batch_pad = jnp.concatenate(
        [b32, jnp.full((N_POOL - N_NODES,), NUM_GRAPHS, jnp.int32)])
    wins, firsts = _pool_sc(h_aug, batch_pad)
    return pl.pallas_call(
        _merge_final_tc,
        grid=(_NSUB,),
        in_specs=[
            pl.BlockSpec(memory_space=pltpu.SMEM),
            pl.BlockSpec((1, LW, AUG), lambda i: (i, 0, 0)),
            pl.BlockSpec((HID, 1), lambda i: (0, 0)),
            pl.BlockSpec(memory_space=pltpu.SMEM),
        ],
        out_specs=pl.BlockSpec((NUM_GRAPHS, 1), lambda i: (0, 0)),
        out_shape=jax.ShapeDtypeStruct((NUM_GRAPHS, 1), jnp.float32),
        scratch_shapes=[pltpu.VMEM((G_PAD2, AUG), jnp.float32)],
    )(firsts, wins, fcW, fcb)


def kernel(x, edge_index, batch, emb, W1, att_src1, att_dst1, b1,
           W2, att_src2, att_dst2, b2, fcW, fcb):
    N = x.shape[0]
    loop = jnp.arange(N, dtype=jnp.int32)
    src = jnp.concatenate([edge_index[0].astype(jnp.int32), loop])
    dst = jnp.concatenate([edge_index[1].astype(jnp.int32), loop])
    dsts, srcs = lax.sort_key_val(dst, src)
    dsts = jnp.concatenate(
        [dsts, jnp.full((E_PAD - E_TOT,), N_TAB - 1, jnp.int32)])
    srcs = jnp.concatenate([srcs, jnp.zeros((E_PAD - E_TOT,), jnp.int32)])

    x_pad = jnp.concatenate(
        [x.astype(jnp.int32), jnp.zeros((N_TAB - N,), jnp.int32)])
    emb128 = jnp.pad(emb, ((0, 0), (0, TROW - EMB)))
    hx = _gather_rows_sc(emb128, x_pad, TROW)

    t1, td1 = _prep1_tc(hx, W1, att_src1, att_dst1)
    t1 = t1.reshape(4 * N_TAB, TROW)
    td1p = jnp.pad(td1, ((0, WIN), (0, 0))).reshape((N_TAB + WIN) // 8, TROW)
    wins1, dens1, firsts1 = _edge_pass_sc(srcs, dsts, t1, td1p, 4)
    num1p, den1p = _merge_tc(wins1, dens1, firsts1, 4)
    num1 = num1p.reshape(4, N_MERGE, HID)
    den1 = den1p.reshape(N_MERGE, _L)

    t2, td2 = _combine1_prep2_tc(
        num1[:, :N_TAB, :], den1[:N_TAB], b1.reshape(1, -1), W2,
        att_src2, att_dst2)
    td2p = jnp.pad(td2, ((0, WIN), (0, 0))).reshape((N_TAB + WIN) // 8, TROW)
    wins2, dens2, firsts2 = _edge_pass_sc(srcs, dsts, t2, td2p, 1)
    num2p, den2p = _merge_tc(wins2, dens2, firsts2, 1)
    num2 = num2p.reshape(1, N_MERGE, HID)
    den2 = den2p.reshape(N_MERGE, _L)

    h_aug_core = _combine2_tc(num2[:, :N_TAB, :], den2[:N_TAB], b2.reshape(1, -1))
    return _pool_and_head(h_aug_core, batch, fcW, fcb)
